# baseline (speedup 1.0000x reference)
import jax
import jax.numpy as jnp
from jax.experimental import pallas as pl

VOCAB = 1000
ROWS = 4096 * 20
BLOCK = 512


def _onehot_block(x_ref, o_ref):
    xb = x_ref[0, 0, :]
    iota = jax.lax.broadcasted_iota(jnp.int32, (BLOCK, VOCAB), 1)
    o_ref[...] = (xb[:, None] == iota).astype(jnp.float32)


def kernel(x):
    n_blocks = ROWS // BLOCK
    x_flat = x.reshape(n_blocks, 1, BLOCK)
    out = pl.pallas_call(
        _onehot_block,
        grid=(n_blocks,),
        in_specs=[pl.BlockSpec((1, 1, BLOCK), lambda i: (i, 0, 0))],
        out_specs=pl.BlockSpec((BLOCK, VOCAB), lambda i: (i, 0)),
        out_shape=jax.ShapeDtypeStruct((ROWS, VOCAB), jnp.float32),
    )(x_flat)
    return out.reshape(x.shape[0], x.shape[1], VOCAB)


# speedup vs baseline: 1.5752x; 1.5752x over previous
import jax
import jax.numpy as jnp
from jax.experimental import pallas as pl

VOCAB = 1000
B0 = 32


def _onehot_block(x_ref, o_ref):
    xb = x_ref[...]
    iota = jax.lax.broadcasted_iota(jnp.int32, (B0, 20, VOCAB), 2)
    o_ref[...] = (xb[:, :, None] == iota).astype(jnp.float32)


def kernel(x):
    n0, n1 = x.shape
    return pl.pallas_call(
        _onehot_block,
        grid=(n0 // B0,),
        in_specs=[pl.BlockSpec((B0, n1), lambda i: (i, 0))],
        out_specs=pl.BlockSpec((B0, n1, VOCAB), lambda i: (i, 0, 0)),
        out_shape=jax.ShapeDtypeStruct((n0, n1, VOCAB), jnp.float32),
    )(x)


# speedup vs baseline: 1.6365x; 1.0389x over previous
import jax
import jax.numpy as jnp
from jax.experimental import pallas as pl
from jax.experimental.pallas import tpu as pltpu

VOCAB = 1000
B0 = 128


def _onehot_block(x_ref, o_ref):
    xb = x_ref[...]
    iota = jax.lax.broadcasted_iota(jnp.int32, (B0, 20, VOCAB), 2)
    o_ref[...] = (xb[:, :, None] == iota).astype(jnp.float32)


def kernel(x):
    n0, n1 = x.shape
    return pl.pallas_call(
        _onehot_block,
        grid=(n0 // B0,),
        in_specs=[pl.BlockSpec((B0, n1), lambda i: (i, 0))],
        out_specs=pl.BlockSpec((B0, n1, VOCAB), lambda i: (i, 0, 0)),
        out_shape=jax.ShapeDtypeStruct((n0, n1, VOCAB), jnp.float32),
        compiler_params=pltpu.CompilerParams(
            dimension_semantics=("parallel",),
            vmem_limit_bytes=100 * 1024 * 1024,
        ),
    )(x)
